# baseline (speedup 1.0000x reference)
import functools

import jax
import jax.numpy as jnp
from jax import lax
from jax.experimental import pallas as pl
from jax.experimental.pallas import tpu as pltpu
from jax.experimental.pallas import tpu_sc as plsc

N = 10000
E = 320000
D = 128

NC = 2
NS = 16
NW = NC * NS
EPW = E // NW
CHUNK = 100
NCHUNK = EPW // CHUNK
IBLK = 10
NBLK = NCHUNK // IBLK

ZROWS = 640
NP = NS * ZROWS

TAIL_OFF = 84
TAIL_SKIP = 12


def _make_sc_agg(do_counts):
    def body(table_hbm, src_hbm, dst_hbm, zeros_hbm, *rest):
        if do_counts:
            (out_hbm, cnt_hbm, si0, si1, di0, di1, rows, rows2, hist,
             acc, semi0, semi1, sem, sem2, sems, sems2) = rest
        else:
            (out_hbm, si0, si1, di0, di1, rows, rows2,
             acc, semi0, semi1, sem, sem2, sems, sems2) = rest
        cid = lax.axis_index("c")
        tid = lax.axis_index("s")
        w = cid * NS + tid
        base = w * NBLK

        pltpu.sync_copy(zeros_hbm, acc.at[pl.ds(tid * ZROWS, ZROWS)])

        if do_counts:
            def zero_hist(j, carry):
                hist[pl.ds(j * 16, 16)] = jnp.zeros((16,), jnp.float32)
                return carry

            lax.fori_loop(0, NP // 16, zero_hist, 0)

            tail_lanes = lax.iota(jnp.int32, 16) >= TAIL_SKIP

            def count_row(di, j):
                for k in range(CHUNK // 16):
                    d = di[j, pl.ds(k * 16, 16)]
                    cnt, last = plsc.scan_count(d)
                    plsc.addupdate_scatter(hist, [d],
                                           cnt.astype(jnp.float32),
                                           mask=last)
                d = di[j, pl.ds(TAIL_OFF, 16)]
                cnt, last = plsc.scan_count(d, tail_lanes)
                plsc.addupdate_scatter(hist, [d], cnt.astype(jnp.float32),
                                       mask=last & tail_lanes)
        else:
            def count_row(di, j):
                pass

        plsc.subcore_barrier()

        def idx_start(b, si, di, s):
            pltpu.make_async_copy(src_hbm.at[base + b], si, s).start()
            pltpu.make_async_copy(dst_hbm.at[base + b], di, s).start()

        def idx_wait(si, di, s):
            pltpu.make_async_copy(src_hbm.at[base], si, s).wait()
            pltpu.make_async_copy(dst_hbm.at[base], di, s).wait()

        def gather(si, j, buf, s):
            return pltpu.make_async_copy(table_hbm.at[si.at[j]], buf, s)

        def scat_start(buf, di, j, s):
            pltpu.async_copy(buf, acc.at[di.at[j]], s, add=True)

        def scat_wait(buf, di, s):
            pltpu.make_async_copy(buf, acc.at[di.at[0]], s).wait()

        idx_start(0, si0, di0, semi0)
        idx_wait(si0, di0, semi0)
        gather(si0, 0, rows, sem).start()
        gather(si0, 1, rows2, sem2).start()

        bufs = ((si0, di0, semi0), (si1, di1, semi1))
        for b in range(NBLK):
            si, di, semi = bufs[b % 2]
            nsi, ndi, nsemi = bufs[(b + 1) % 2]
            if b + 1 < NBLK:
                idx_start(b + 1, nsi, ndi, nsemi)

            def chunk_body(jj, carry, si=si, di=di):
                a = 2 * jj
                gather(si, 0, rows, sem).wait()
                scat_start(rows, di, a, sems)
                count_row(di, a)
                gather(si, 0, rows2, sem2).wait()
                scat_start(rows2, di, a + 1, sems2)
                count_row(di, a + 1)
                scat_wait(rows, di, sems)
                gather(si, jnp.minimum(a + 2, IBLK - 1), rows, sem).start()
                scat_wait(rows2, di, sems2)
                gather(si, jnp.minimum(a + 3, IBLK - 1), rows2, sem2).start()
                return carry

            lax.fori_loop(0, IBLK // 2, chunk_body, 0)
            gather(si, 0, rows, sem).wait()
            gather(si, 0, rows2, sem2).wait()
            if b + 1 < NBLK:
                idx_wait(nsi, ndi, nsemi)
                gather(nsi, 0, rows, sem).start()
                gather(nsi, 1, rows2, sem2).start()


        if do_counts:
            pltpu.sync_copy(hist, cnt_hbm.at[w])

        plsc.subcore_barrier()

        pltpu.sync_copy(acc.at[pl.ds(tid * ZROWS, ZROWS)],
                        out_hbm.at[cid, pl.ds(tid * ZROWS, ZROWS)])

    out_types = [jax.ShapeDtypeStruct((NC, NP, D), jnp.float32)]
    scratch = [
        pltpu.VMEM((IBLK, CHUNK), jnp.int32),
        pltpu.VMEM((IBLK, CHUNK), jnp.int32),
        pltpu.VMEM((IBLK, CHUNK), jnp.int32),
        pltpu.VMEM((IBLK, CHUNK), jnp.int32),
        pltpu.VMEM((CHUNK, D), jnp.float32),
        pltpu.VMEM((CHUNK, D), jnp.float32),
    ]
    if do_counts:
        out_types.append(jax.ShapeDtypeStruct((NW, NP), jnp.float32))
        scratch.append(pltpu.VMEM((NP,), jnp.float32))
    scratch.append(pltpu.VMEM_SHARED((NP, D), jnp.float32))
    scratch.append(pltpu.SemaphoreType.DMA)
    scratch.append(pltpu.SemaphoreType.DMA)
    scratch.append(pltpu.SemaphoreType.DMA)
    scratch.append(pltpu.SemaphoreType.DMA)
    scratch.append(pltpu.SemaphoreType.DMA)
    scratch.append(pltpu.SemaphoreType.DMA)
    return pl.kernel(
        body,
        out_type=tuple(out_types) if do_counts else out_types[0],
        mesh=plsc.VectorSubcoreMesh(core_axis_name="c", subcore_axis_name="s"),
        compiler_params=pltpu.CompilerParams(needs_layout_passes=False),
        scratch_types=scratch,
    )


_sc_agg_counts = _make_sc_agg(True)
_sc_agg_plain = _make_sc_agg(False)


ROWS_BLK = 1024


def _dense_body(relu, p_ref, cnt_ref, xin_ref, wl_ref, bl_ref, wr_ref,
                out_ref):
    p = p_ref[...]
    s = p[0] + p[1]
    ones_col = jnp.ones((NW, 1), jnp.float32)
    c = lax.dot_general(cnt_ref[...], ones_col, (((0,), (0,)), ((), ())),
                        preferred_element_type=jnp.float32)
    cnt = jnp.maximum(c, 1.0)
    mean = s / cnt
    h = (jnp.dot(mean, wl_ref[...], preferred_element_type=jnp.float32)
         + bl_ref[...]
         + jnp.dot(xin_ref[...], wr_ref[...],
                   preferred_element_type=jnp.float32))
    if relu:
        h = jnp.maximum(h, 0.0)
    out_ref[...] = h


def _dense(p, cnt, xin, wl, bl, wr, relu):
    return pl.pallas_call(
        functools.partial(_dense_body, relu),
        grid=(NP // ROWS_BLK,),
        in_specs=[
            pl.BlockSpec((NC, ROWS_BLK, D), lambda i: (0, i, 0)),
            pl.BlockSpec((NW, ROWS_BLK), lambda i: (0, i)),
            pl.BlockSpec((ROWS_BLK, D), lambda i: (i, 0)),
            pl.BlockSpec((D, D), lambda i: (0, 0)),
            pl.BlockSpec((1, D), lambda i: (0, 0)),
            pl.BlockSpec((D, D), lambda i: (0, 0)),
        ],
        out_specs=pl.BlockSpec((ROWS_BLK, D), lambda i: (i, 0)),
        out_shape=jax.ShapeDtypeStruct((NP, D), jnp.float32),
    )(p, cnt, xin, wl, bl, wr)


def kernel(x, edge_index, W1l, b1l, W1r, W2l, b2l, W2r):
    src = edge_index[0].astype(jnp.int32)
    dst = edge_index[1].astype(jnp.int32)
    src2d = src.reshape(NW * NBLK, IBLK, CHUNK)
    dst2d = dst.reshape(NW * NBLK, IBLK, CHUNK)
    zrows = jnp.zeros((ZROWS, D), jnp.float32)
    xp = jnp.pad(x, ((0, NP - N), (0, 0)))

    p1, cnt = _sc_agg_counts(xp, src2d, dst2d, zrows)
    h = _dense(p1, cnt, xp, W1l, b1l.reshape(1, D), W1r, relu=True)
    p2 = _sc_agg_plain(h, src2d, dst2d, zrows)
    out = _dense(p2, cnt, h, W2l, b2l.reshape(1, D), W2r, relu=False)
    return (out[:N], out[:N], out[:N], out[:N])


# speedup vs baseline: 1.0575x; 1.0575x over previous
import functools

import jax
import jax.numpy as jnp
from jax import lax
from jax.experimental import pallas as pl
from jax.experimental.pallas import tpu as pltpu
from jax.experimental.pallas import tpu_sc as plsc

N = 10000
E = 320000
D = 128

NC = 2
NS = 16
NW = NC * NS
EPW = E // NW
CHUNK = 100
NCHUNK = EPW // CHUNK
IBLK = 10
NBLK = NCHUNK // IBLK

ZROWS = 640
NP = NS * ZROWS

TAIL_OFF = 84
TAIL_SKIP = 12


def _make_sc_agg(do_counts):
    def body(table_hbm, src_hbm, dst_hbm, zeros_hbm, *rest):
        if do_counts:
            (out_hbm, cnt_hbm, si0, si1, di0, di1, rows, rows2, hist,
             acc, semi0, semi1, sem, sem2, sems, sems2) = rest
        else:
            (out_hbm, si0, si1, di0, di1, rows, rows2,
             acc, semi0, semi1, sem, sem2, sems, sems2) = rest
        cid = lax.axis_index("c")
        tid = lax.axis_index("s")
        w = cid * NS + tid
        base = w * NBLK

        pltpu.sync_copy(zeros_hbm, acc.at[pl.ds(tid * ZROWS, ZROWS)])

        if do_counts:
            def zero_hist(j, carry):
                hist[pl.ds(j * 16, 16)] = jnp.zeros((16,), jnp.float32)
                return carry

            lax.fori_loop(0, NP // 16, zero_hist, 0)

            tail_lanes = lax.iota(jnp.int32, 16) >= TAIL_SKIP

            def count_row(di, j):
                for k in range(CHUNK // 16):
                    d = di[j, pl.ds(k * 16, 16)]
                    cnt, last = plsc.scan_count(d)
                    plsc.addupdate_scatter(hist, [d],
                                           cnt.astype(jnp.float32),
                                           mask=last)
                d = di[j, pl.ds(TAIL_OFF, 16)]
                cnt, last = plsc.scan_count(d, tail_lanes)
                plsc.addupdate_scatter(hist, [d], cnt.astype(jnp.float32),
                                       mask=last & tail_lanes)
        else:
            def count_row(di, j):
                pass

        plsc.subcore_barrier()

        def idx_start(b, si, di, s):
            pltpu.make_async_copy(src_hbm.at[base + b], si, s).start()
            pltpu.make_async_copy(dst_hbm.at[base + b], di, s).start()

        def idx_wait(si, di, s):
            pltpu.make_async_copy(src_hbm.at[base], si, s).wait()
            pltpu.make_async_copy(dst_hbm.at[base], di, s).wait()

        def gather(si, j, buf, s):
            return pltpu.make_async_copy(table_hbm.at[si.at[j]], buf, s)

        idx_start(0, si0, di0, semi0)
        idx_wait(si0, di0, semi0)
        gather(si0, 0, rows, sem).start()

        bufs = ((si0, di0, semi0), (si1, di1, semi1))
        for b in range(NBLK):
            si, di, semi = bufs[b % 2]
            nsi, ndi, nsemi = bufs[(b + 1) % 2]
            if b + 1 < NBLK:
                idx_start(b + 1, nsi, ndi, nsemi)

            def chunk_body(jj, carry, si=si, di=di):
                a = 2 * jj
                gather(si, a, rows, sem).wait()
                gather(si, a + 1, rows2, sem2).start()
                count_row(di, a)
                pltpu.sync_copy(rows, acc.at[di.at[a]], add=True)
                gather(si, 0, rows2, sem2).wait()
                nxt = jnp.minimum(a + 2, IBLK - 1)
                gather(si, nxt, rows, sem).start()
                count_row(di, a + 1)
                pltpu.sync_copy(rows2, acc.at[di.at[a + 1]], add=True)
                return carry

            lax.fori_loop(0, IBLK // 2, chunk_body, 0)
            gather(si, 0, rows, sem).wait()
            if b + 1 < NBLK:
                idx_wait(nsi, ndi, nsemi)
                gather(nsi, 0, rows, sem).start()


        if do_counts:
            pltpu.sync_copy(hist, cnt_hbm.at[w])

        plsc.subcore_barrier()

        pltpu.sync_copy(acc.at[pl.ds(tid * ZROWS, ZROWS)],
                        out_hbm.at[cid, pl.ds(tid * ZROWS, ZROWS)])

    out_types = [jax.ShapeDtypeStruct((NC, NP, D), jnp.float32)]
    scratch = [
        pltpu.VMEM((IBLK, CHUNK), jnp.int32),
        pltpu.VMEM((IBLK, CHUNK), jnp.int32),
        pltpu.VMEM((IBLK, CHUNK), jnp.int32),
        pltpu.VMEM((IBLK, CHUNK), jnp.int32),
        pltpu.VMEM((CHUNK, D), jnp.float32),
        pltpu.VMEM((CHUNK, D), jnp.float32),
    ]
    if do_counts:
        out_types.append(jax.ShapeDtypeStruct((NW, NP), jnp.float32))
        scratch.append(pltpu.VMEM((NP,), jnp.float32))
    scratch.append(pltpu.VMEM_SHARED((NP, D), jnp.float32))
    scratch.append(pltpu.SemaphoreType.DMA)
    scratch.append(pltpu.SemaphoreType.DMA)
    scratch.append(pltpu.SemaphoreType.DMA)
    scratch.append(pltpu.SemaphoreType.DMA)
    scratch.append(pltpu.SemaphoreType.DMA)
    scratch.append(pltpu.SemaphoreType.DMA)
    return pl.kernel(
        body,
        out_type=tuple(out_types) if do_counts else out_types[0],
        mesh=plsc.VectorSubcoreMesh(core_axis_name="c", subcore_axis_name="s"),
        compiler_params=pltpu.CompilerParams(needs_layout_passes=False),
        scratch_types=scratch,
    )


_sc_agg_counts = _make_sc_agg(True)
_sc_agg_plain = _make_sc_agg(False)


ROWS_BLK = 1024


def _dense_body(relu, p_ref, cnt_ref, xin_ref, wl_ref, bl_ref, wr_ref,
                out_ref):
    p = p_ref[...]
    s = p[0] + p[1]
    ones_col = jnp.ones((NW, 1), jnp.float32)
    c = lax.dot_general(cnt_ref[...], ones_col, (((0,), (0,)), ((), ())),
                        preferred_element_type=jnp.float32)
    cnt = jnp.maximum(c, 1.0)
    mean = s / cnt
    h = (jnp.dot(mean, wl_ref[...], preferred_element_type=jnp.float32)
         + bl_ref[...]
         + jnp.dot(xin_ref[...], wr_ref[...],
                   preferred_element_type=jnp.float32))
    if relu:
        h = jnp.maximum(h, 0.0)
    out_ref[...] = h


def _dense(p, cnt, xin, wl, bl, wr, relu):
    return pl.pallas_call(
        functools.partial(_dense_body, relu),
        grid=(NP // ROWS_BLK,),
        in_specs=[
            pl.BlockSpec((NC, ROWS_BLK, D), lambda i: (0, i, 0)),
            pl.BlockSpec((NW, ROWS_BLK), lambda i: (0, i)),
            pl.BlockSpec((ROWS_BLK, D), lambda i: (i, 0)),
            pl.BlockSpec((D, D), lambda i: (0, 0)),
            pl.BlockSpec((1, D), lambda i: (0, 0)),
            pl.BlockSpec((D, D), lambda i: (0, 0)),
        ],
        out_specs=pl.BlockSpec((ROWS_BLK, D), lambda i: (i, 0)),
        out_shape=jax.ShapeDtypeStruct((NP, D), jnp.float32),
    )(p, cnt, xin, wl, bl, wr)


def kernel(x, edge_index, W1l, b1l, W1r, W2l, b2l, W2r):
    src = edge_index[0].astype(jnp.int32)
    dst = edge_index[1].astype(jnp.int32)
    src2d = src.reshape(NW * NBLK, IBLK, CHUNK)
    dst2d = dst.reshape(NW * NBLK, IBLK, CHUNK)
    zrows = jnp.zeros((ZROWS, D), jnp.float32)
    xp = jnp.pad(x, ((0, NP - N), (0, 0)))

    p1, cnt = _sc_agg_counts(xp, src2d, dst2d, zrows)
    h = _dense(p1, cnt, xp, W1l, b1l.reshape(1, D), W1r, relu=True)
    p2 = _sc_agg_plain(h, src2d, dst2d, zrows)
    out = _dense(p2, cnt, h, W2l, b2l.reshape(1, D), W2r, relu=False)
    return (out[:N], out[:N], out[:N], out[:N])
